# baseline (speedup 1.0000x reference)
import jax
import jax.numpy as jnp
from jax.experimental import pallas as pl
from jax.experimental.pallas import tpu as pltpu

_EPS = 1e-5
_FP8_MAX = 448.0


def _fused_body(x_ref, nw_ref, w_ref, sin_ref, sout_ref, o_ref, q_scr):
    h = x_ref.shape[-1]
    bm = x_ref.shape[0]
    r_in = 1.0 / sin_ref[0, 0]
    nw = nw_ref[...].astype(jnp.float32)
    ch = 128
    for r in range(0, bm, ch):
        xf = x_ref[r:r + ch, :].astype(jnp.float32)
        ssq = jnp.sum(xf * xf, axis=-1, keepdims=True)
        inv_rms = jax.lax.rsqrt(ssq * (1.0 / h) + _EPS)
        normed = (xf * (inv_rms * r_in)) * nw
        q_scr[r:r + ch, :] = jax.lax.clamp(
            -_FP8_MAX, normed, _FP8_MAX).astype(jnp.float8_e4m3fn)
    acc = jax.lax.dot_general(
        q_scr[...], w_ref[...],
        dimension_numbers=(((1,), (1,)), ((), ())),
        preferred_element_type=jnp.float32,
    )
    o_ref[...] = (acc * sout_ref[0, 0]).astype(jnp.bfloat16)


def kernel(x, norm_weight, weight_fp8, input_scale, weight_scale):
    t, h = x.shape
    o = weight_fp8.shape[0]
    bm = 512
    nw2d = norm_weight.reshape(1, h)
    sin = jnp.reshape(input_scale.astype(jnp.float32), (1, 1))
    sout = jnp.reshape((input_scale * weight_scale).astype(jnp.float32), (1, 1))
    return pl.pallas_call(
        _fused_body,
        grid=(t // bm,),
        in_specs=[
            pl.BlockSpec((bm, h), lambda i: (i, 0)),
            pl.BlockSpec((1, h), lambda i: (0, 0)),
            pl.BlockSpec((o, h), lambda i: (0, 0)),
            pl.BlockSpec(memory_space=pltpu.SMEM),
            pl.BlockSpec(memory_space=pltpu.SMEM),
        ],
        out_specs=pl.BlockSpec((bm, o), lambda i: (i, 0)),
        out_shape=jax.ShapeDtypeStruct((t, o), jnp.bfloat16),
        scratch_shapes=[pltpu.VMEM((bm, h), jnp.float8_e4m3fn)],
        compiler_params=pltpu.CompilerParams(
            dimension_semantics=("parallel",),
            vmem_limit_bytes=56 * 1024 * 1024,
        ),
        name="rmsnorm_quant_fp8_gemm",
    )(x, nw2d, weight_fp8, sin, sout)
